# baseline (speedup 1.0000x reference)
import jax
import jax.numpy as jnp
from jax import lax
from jax.experimental import pallas as pl
from jax.experimental.pallas import tpu as pltpu
from jax.experimental.pallas import tpu_sc as plsc

VOCAB = 100000
D = 64
B = 16384
CTX = 20

NC = 2
NS = 16
NW = NC * NS
BW = B // NW
G = 32
ROWS = G * CTX
NCH = ROWS // 128
NG = BW // G
PW = BW * CTX


def _sc_body(cen_ids_hbm, ctx_ids_hbm, in_hbm, out_hbm, scores_hbm,
             cen_idx_v, ctx_raw_v, ctx_idx_v, cen_rows_v, ctx_rows_v,
             scores_v, sem0, sem1):
    wid = lax.axis_index("s") * NC + lax.axis_index("c")

    pltpu.sync_copy(cen_ids_hbm.at[pl.ds(wid * BW, BW)], cen_idx_v)
    pltpu.sync_copy(ctx_ids_hbm.at[pl.ds(wid * BW, BW), :], ctx_raw_v)

    lane = lax.iota(jnp.int32, 16)
    mask15 = lane == 15
    l0 = lane * 0

    @plsc.parallel_loop(0, PW // 16, unroll=2)
    def _flat(j):
        pos = j * 16 + lane
        row = (pos * 52429) >> 20
        col = pos - row * CTX
        vals = plsc.load_gather(ctx_raw_v, [row, col])
        ctx_idx_v[pl.ds(j * 16, 16)] = vals

    sems = (sem0, sem1)

    def _descs(g, b):
        sem = sems[b]
        ds = []
        for k in range(NCH):
            ds.append(pltpu.make_async_copy(
                out_hbm.at[ctx_idx_v.at[pl.ds((g * NCH + k) * 128, 128)]],
                ctx_rows_v.at[b, pl.ds(k * 128, 128)],
                sem))
        ds.append(pltpu.make_async_copy(
            in_hbm.at[cen_idx_v.at[pl.ds(g * G, G)]],
            cen_rows_v.at[b],
            sem))
        return ds

    def _issue(g, b):
        for d in _descs(g, b):
            d.start()

    def _wait(g, b):
        for d in _descs(g, b):
            d.wait()

    def _compute(g, b):
        @plsc.parallel_loop(0, G, unroll=2)
        def _item(i):
            cen = [cen_rows_v[b, i, pl.ds(16 * k, 16)] for k in range(4)]
            base = i * CTX
            gbase = (g * G + i) * CTX
            for c in range(CTX):
                r = base + c
                p = ctx_rows_v[b, r, pl.ds(0, 16)] * cen[0]
                p += ctx_rows_v[b, r, pl.ds(16, 16)] * cen[1]
                p += ctx_rows_v[b, r, pl.ds(32, 16)] * cen[2]
                p += ctx_rows_v[b, r, pl.ds(48, 16)] * cen[3]
                s = plsc.cumsum(p)
                plsc.store_scatter(scores_v, [gbase + c + l0], s,
                                   mask=mask15)

    _issue(0, 0)
    _issue(1, 1)

    @pl.loop(0, NG, step=2)
    def _group(g):
        for b in range(2):
            gg = g + b
            _wait(gg, b)

            @pl.when(gg + 2 < NG)
            def _():
                _issue(gg + 2, b)

            _compute(gg, b)

    pltpu.sync_copy(scores_v, scores_hbm.at[wid])


def _tc_loss_body(s_ref, o_ref):
    s = s_ref[...]
    sp = jnp.maximum(-s, 0.0) + jnp.log1p(jnp.exp(-jnp.abs(s)))
    o_ref[...] = (jnp.sum(sp) / jnp.float32(B * CTX)).reshape(1, 1)


@jax.jit
def kernel(center_ids, context_ids, in_embed, out_embed):
    cen_ids = center_ids.astype(jnp.int32)
    ctx_ids = context_ids.astype(jnp.int32)

    mesh = plsc.VectorSubcoreMesh(core_axis_name="c", subcore_axis_name="s")
    scores = pl.kernel(
        _sc_body,
        out_type=jax.ShapeDtypeStruct((NW, PW), jnp.float32),
        mesh=mesh,
        compiler_params=pltpu.CompilerParams(
            needs_layout_passes=False, use_tc_tiling_on_sc=False),
        scratch_types=[
            pltpu.VMEM((BW,), jnp.int32),
            pltpu.VMEM((BW, CTX), jnp.int32),
            pltpu.VMEM((PW,), jnp.int32),
            pltpu.VMEM((2, G, D), jnp.float32),
            pltpu.VMEM((2, ROWS, D), jnp.float32),
            pltpu.VMEM((PW,), jnp.float32),
            pltpu.SemaphoreType.DMA,
            pltpu.SemaphoreType.DMA,
        ],
    )(cen_ids, ctx_ids, in_embed, out_embed)

    loss = pl.pallas_call(
        _tc_loss_body,
        out_shape=jax.ShapeDtypeStruct((1, 1), jnp.float32),
    )(scores)
    return loss[0, 0]


# speedup vs baseline: 1.1565x; 1.1565x over previous
import jax
import jax.numpy as jnp
from jax import lax
from jax.experimental import pallas as pl
from jax.experimental.pallas import tpu as pltpu
from jax.experimental.pallas import tpu_sc as plsc

VOCAB = 100000
D = 64
B = 16384
CTX = 20

NC = 2
NS = 16
NW = NC * NS
BW = B // NW
G = 16
ROWS = G * CTX
CH = 64
NCH = ROWS // CH
NG = BW // G
PW = BW * CTX


def _sc_body(cen_ids_hbm, ctx_ids_hbm, in_hbm, out_hbm, scores_hbm,
             cen_idx_v, ctx_idx_v, cen_rows_v, ctx_rows_v,
             scores_v, sem0, sem1):
    wid = lax.axis_index("s") * NC + lax.axis_index("c")

    pltpu.sync_copy(cen_ids_hbm.at[pl.ds(wid * BW, BW)], cen_idx_v)
    pltpu.sync_copy(ctx_ids_hbm.at[pl.ds(wid * BW * CTX, BW * CTX)], ctx_idx_v)

    lane = lax.iota(jnp.int32, 16)
    mask15 = lane == 15
    l0 = lane * 0

    sems = (sem0, sem1)

    def _descs(g, b):
        sem = sems[b]
        ds = []
        for k in range(NCH):
            ds.append(pltpu.make_async_copy(
                out_hbm.at[ctx_idx_v.at[pl.ds((g * NCH + k) * CH, CH)]],
                ctx_rows_v.at[b, pl.ds(k * CH, CH)],
                sem))
        ds.append(pltpu.make_async_copy(
            in_hbm.at[cen_idx_v.at[pl.ds(g * G, G)]],
            cen_rows_v.at[b],
            sem))
        return ds

    def _issue(g, b):
        for d in _descs(g, b):
            d.start()

    def _wait(g, b):
        for d in _descs(g, b):
            d.wait()

    def _compute(g, b):
        @plsc.parallel_loop(0, G, unroll=2)
        def _item(i):
            cen = [cen_rows_v[b, i, pl.ds(16 * k, 16)] for k in range(4)]
            base = i * CTX
            iv = (g * G + i) * CTX + l0
            for c in range(CTX):
                r = base + c
                p = ctx_rows_v[b, r, pl.ds(0, 16)] * cen[0]
                p += ctx_rows_v[b, r, pl.ds(16, 16)] * cen[1]
                p += ctx_rows_v[b, r, pl.ds(32, 16)] * cen[2]
                p += ctx_rows_v[b, r, pl.ds(48, 16)] * cen[3]
                s = plsc.cumsum(p)
                plsc.store_scatter(scores_v, [iv + c], s, mask=mask15)

    _issue(0, 0)
    _issue(1, 1)

    @pl.loop(0, NG, step=2)
    def _group(g):
        for b in range(2):
            gg = g + b
            _wait(gg, b)

            @pl.when(gg + 2 < NG)
            def _():
                _issue(gg + 2, b)

            _compute(gg, b)

    pltpu.sync_copy(scores_v, scores_hbm.at[wid])


def _tc_loss_body(s_ref, o_ref):
    s = s_ref[...]
    sp = jnp.maximum(-s, 0.0) + jnp.log1p(jnp.exp(-jnp.abs(s)))
    o_ref[...] = (jnp.sum(sp) / jnp.float32(B * CTX)).reshape(1, 1)


@jax.jit
def kernel(center_ids, context_ids, in_embed, out_embed):
    cen_ids = center_ids.astype(jnp.int32)
    ctx_ids = context_ids.astype(jnp.int32).reshape(-1)

    mesh = plsc.VectorSubcoreMesh(core_axis_name="c", subcore_axis_name="s")
    scores = pl.kernel(
        _sc_body,
        out_type=jax.ShapeDtypeStruct((NW, PW), jnp.float32),
        mesh=mesh,
        compiler_params=pltpu.CompilerParams(
            needs_layout_passes=False, use_tc_tiling_on_sc=False),
        scratch_types=[
            pltpu.VMEM((BW,), jnp.int32),
            pltpu.VMEM((BW * CTX,), jnp.int32),
            pltpu.VMEM((2, G, D), jnp.float32),
            pltpu.VMEM((2, ROWS, D), jnp.float32),
            pltpu.VMEM((PW,), jnp.float32),
            pltpu.SemaphoreType.DMA,
            pltpu.SemaphoreType.DMA,
        ],
    )(cen_ids, ctx_ids, in_embed, out_embed)

    loss = pl.pallas_call(
        _tc_loss_body,
        out_shape=jax.ShapeDtypeStruct((1, 1), jnp.float32),
    )(scores)
    return loss[0, 0]


# speedup vs baseline: 1.2818x; 1.1084x over previous
import jax
import jax.numpy as jnp
from jax import lax
from jax.experimental import pallas as pl
from jax.experimental.pallas import tpu as pltpu
from jax.experimental.pallas import tpu_sc as plsc

VOCAB = 100000
D = 64
B = 16384
CTX = 20

NC = 2
NS = 16
NW = NC * NS
BW = B // NW
G = 32
ROWS = G * CTX
CH = 128
NCH = ROWS // CH
NG = BW // G
PW = BW * CTX


def _sc_body(cen_ids_hbm, ctx_ids_hbm, in_hbm, out_hbm, scores_hbm,
             cen_idx_v, ctx_idx_v, cen_rows_v, ctx_rows_v,
             scores_v, sem0, sem1):
    wid = lax.axis_index("s") * NC + lax.axis_index("c")

    pltpu.sync_copy(cen_ids_hbm.at[pl.ds(wid * BW, BW)], cen_idx_v)
    pltpu.sync_copy(ctx_ids_hbm.at[pl.ds(wid * BW * CTX, BW * CTX)], ctx_idx_v)

    lane = lax.iota(jnp.int32, 16)
    mask15 = lane == 15
    l0 = lane * 0

    sems = (sem0, sem1)

    def _descs(g, b):
        sem = sems[b]
        ds = []
        for k in range(NCH):
            ds.append(pltpu.make_async_copy(
                out_hbm.at[ctx_idx_v.at[pl.ds((g * NCH + k) * CH, CH)]],
                ctx_rows_v.at[b, pl.ds(k * CH, CH)],
                sem))
        ds.append(pltpu.make_async_copy(
            in_hbm.at[cen_idx_v.at[pl.ds(g * G, G)]],
            cen_rows_v.at[b],
            sem))
        return ds

    def _issue(g, b):
        for d in _descs(g, b):
            d.start()

    def _wait(g, b):
        for d in _descs(g, b):
            d.wait()

    def _compute(g, b):
        @plsc.parallel_loop(0, G, unroll=2)
        def _item(i):
            cen = [cen_rows_v[b, i, pl.ds(16 * k, 16)] for k in range(4)]
            base = i * CTX
            iv = (g * G + i) * CTX + l0
            for c in range(CTX):
                r = base + c
                p = ctx_rows_v[b, r, pl.ds(0, 16)] * cen[0]
                p += ctx_rows_v[b, r, pl.ds(16, 16)] * cen[1]
                p += ctx_rows_v[b, r, pl.ds(32, 16)] * cen[2]
                p += ctx_rows_v[b, r, pl.ds(48, 16)] * cen[3]
                s = plsc.cumsum(p)
                plsc.store_scatter(scores_v, [iv + c], s, mask=mask15)

    _issue(0, 0)
    _issue(1, 1)

    @pl.loop(0, NG, step=2)
    def _group(g):
        for b in range(2):
            gg = g + b
            _wait(gg, b)

            @pl.when(gg + 2 < NG)
            def _():
                _issue(gg + 2, b)

            _compute(gg, b)

    pltpu.sync_copy(scores_v, scores_hbm.at[wid])


def _tc_loss_body(s_ref, o_ref):
    s = s_ref[...]
    sp = jnp.maximum(-s, 0.0) + jnp.log1p(jnp.exp(-jnp.abs(s)))
    o_ref[...] = (jnp.sum(sp) / jnp.float32(B * CTX)).reshape(1, 1)


@jax.jit
def kernel(center_ids, context_ids, in_embed, out_embed):
    cen_ids = center_ids.astype(jnp.int32)
    ctx_ids = context_ids.astype(jnp.int32).reshape(-1)

    mesh = plsc.VectorSubcoreMesh(core_axis_name="c", subcore_axis_name="s")
    scores = pl.kernel(
        _sc_body,
        out_type=jax.ShapeDtypeStruct((NW, PW), jnp.float32),
        mesh=mesh,
        compiler_params=pltpu.CompilerParams(
            needs_layout_passes=False, use_tc_tiling_on_sc=False),
        scratch_types=[
            pltpu.VMEM((BW,), jnp.int32),
            pltpu.VMEM((BW * CTX,), jnp.int32),
            pltpu.VMEM((2, G, D), jnp.float32),
            pltpu.VMEM((2, ROWS, D), jnp.float32),
            pltpu.VMEM((PW,), jnp.float32),
            pltpu.SemaphoreType.DMA,
            pltpu.SemaphoreType.DMA,
        ],
    )(cen_ids, ctx_ids, in_embed, out_embed)

    loss = pl.pallas_call(
        _tc_loss_body,
        out_shape=jax.ShapeDtypeStruct((1, 1), jnp.float32),
    )(scores)
    return loss[0, 0]
